# baseline (speedup 1.0000x reference)
import functools

import jax
import jax.numpy as jnp
from jax import lax
from jax.experimental import pallas as pl
from jax.experimental.pallas import tpu as pltpu
from jax.experimental.pallas import tpu_sc as plsc

VOCAB = 1_000_000
EMBED_DIM = 64
BATCH = 4096
HIST = 50
N = BATCH * HIST


RBLK = 512


def _transpose_body(tt_ref, out_ref):
    x = tt_ref[...]
    out_ref[:, 0:EMBED_DIM] = x[:, : RBLK // 2].T
    out_ref[:, EMBED_DIM : 2 * EMBED_DIM] = x[:, RBLK // 2 :].T


_transpose_call = pl.pallas_call(
    _transpose_body,
    grid=(VOCAB // RBLK,),
    in_specs=[pl.BlockSpec((EMBED_DIM, RBLK), lambda i: (0, i))],
    out_specs=pl.BlockSpec((RBLK // 2, 2 * EMBED_DIM), lambda i: (i, 0)),
    out_shape=jax.ShapeDtypeStruct((VOCAB // 2, 2 * EMBED_DIM), jnp.float32),
)


_info = plsc.get_sparse_core_info()
NC = _info.num_cores
NS = _info.num_subcores
NW = NC * NS
PER_W = N // NW
CHUNK = 128
NCHUNK = PER_W // CHUNK
NBUF = 5
NITER = NCHUNK // NBUF

_mesh = plsc.VectorSubcoreMesh(core_axis_name="c", subcore_axis_name="s")


@functools.partial(
    pl.kernel,
    mesh=_mesh,
    compiler_params=pltpu.CompilerParams(use_tc_tiling_on_sc=False),
    out_type=jax.ShapeDtypeStruct((N, EMBED_DIM), jnp.float32),
    scratch_types=(
        [
            pltpu.VMEM((NCHUNK, CHUNK), jnp.int32),
            pltpu.VMEM((NBUF, CHUNK, EMBED_DIM), jnp.float32),
        ]
        + [pltpu.SemaphoreType.DMA] * (2 * NBUF)
    ),
)
def _gather_kernel(table_hbm, idx_hbm, out_hbm, idx_v, rows_v, *sems):
    gsem = sems[:NBUF]
    wsem = sems[NBUF:]
    wid = lax.axis_index("s") * NC + lax.axis_index("c")
    base = wid * PER_W
    pltpu.sync_copy(idx_hbm.at[wid], idx_v)

    def fire_gather(c, b):
        pltpu.async_copy(table_hbm.at[idx_v.at[c]], rows_v.at[b], gsem[b])

    def wait_gather(b):
        pltpu.make_async_copy(table_hbm.at[idx_v.at[0]], rows_v.at[b], gsem[b]).wait()

    def fire_write(c, b):
        pltpu.async_copy(
            rows_v.at[b], out_hbm.at[pl.ds(base + c * CHUNK, CHUNK)], wsem[b]
        )

    def wait_write(b):
        pltpu.make_async_copy(
            rows_v.at[b], out_hbm.at[pl.ds(base, CHUNK)], wsem[b]
        ).wait()

    for b in range(NBUF):
        fire_gather(b, b)

    def cycle(i, carry):
        for b in range(NBUF):
            wait_gather(b)
            fire_write(i * NBUF + b, b)
        for b in range(NBUF):
            wait_write(b)
            fire_gather((i + 1) * NBUF + b, b)
        return carry

    lax.fori_loop(0, NITER - 1, cycle, 0)

    last = (NITER - 1) * NBUF
    for b in range(NBUF):
        wait_gather(b)
        fire_write(last + b, b)
    for b in range(NBUF):
        wait_write(b)


def kernel(encoded_input, table):
    idx = encoded_input.reshape(-1).astype(jnp.int32)
    half = RBLK // 2
    perm = (
        (idx & ~(RBLK - 1))
        + 2 * (idx & (half - 1))
        + ((idx // half) & 1)
    )
    idx3 = perm.reshape(NW, NCHUNK, CHUNK)
    table_rows = _transpose_call(table.T).reshape(VOCAB, EMBED_DIM)
    out = _gather_kernel(table_rows, idx3)
    return out.reshape(BATCH, HIST, EMBED_DIM)


# speedup vs baseline: 1.8541x; 1.8541x over previous
import functools

import jax
import jax.numpy as jnp
from jax import lax
from jax.experimental import pallas as pl
from jax.experimental.pallas import tpu as pltpu
from jax.experimental.pallas import tpu_sc as plsc

VOCAB = 1_000_000
EMBED_DIM = 64
PADDED_DIM = 128
BATCH = 4096
HIST = 50
N = BATCH * HIST

_info = plsc.get_sparse_core_info()
NC = _info.num_cores
NS = _info.num_subcores
NW = NC * NS
PER_W = N // NW
CHUNK = 128
NCHUNK = PER_W // CHUNK
NBUF = 5
NITER = NCHUNK // NBUF

_mesh = plsc.VectorSubcoreMesh(core_axis_name="c", subcore_axis_name="s")


@functools.partial(
    pl.kernel,
    mesh=_mesh,
    compiler_params=pltpu.CompilerParams(use_tc_tiling_on_sc=False),
    out_type=jax.ShapeDtypeStruct((N, EMBED_DIM), jnp.float32),
    scratch_types=(
        [
            pltpu.VMEM((NCHUNK, CHUNK), jnp.int32),
            pltpu.VMEM((NBUF, CHUNK, PADDED_DIM), jnp.float32),
        ]
        + [pltpu.SemaphoreType.DMA] * (2 * NBUF)
    ),
)
def _gather_kernel(table_hbm, idx_hbm, out_hbm, idx_v, rows_v, *sems):
    gsem = sems[:NBUF]
    wsem = sems[NBUF:]
    wid = lax.axis_index("s") * NC + lax.axis_index("c")
    base = wid * PER_W
    pltpu.sync_copy(idx_hbm.at[wid], idx_v)

    def fire_gather(c, b):
        pltpu.async_copy(table_hbm.at[idx_v.at[c]], rows_v.at[b], gsem[b])

    def wait_gather(b):
        pltpu.make_async_copy(table_hbm.at[idx_v.at[0]], rows_v.at[b], gsem[b]).wait()

    def fire_write(c, b):
        pltpu.async_copy(
            rows_v.at[b, :, pl.ds(0, EMBED_DIM)],
            out_hbm.at[pl.ds(base + c * CHUNK, CHUNK)],
            wsem[b],
        )

    def wait_write(b):
        pltpu.make_async_copy(
            rows_v.at[b, :, pl.ds(0, EMBED_DIM)],
            out_hbm.at[pl.ds(base, CHUNK)],
            wsem[b],
        ).wait()

    for b in range(NBUF):
        fire_gather(b, b)

    def cycle(i, carry):
        for b in range(NBUF):
            wait_gather(b)
            fire_write(i * NBUF + b, b)
        for b in range(NBUF):
            wait_write(b)
            fire_gather((i + 1) * NBUF + b, b)
        return carry

    lax.fori_loop(0, NITER - 1, cycle, 0)

    last = (NITER - 1) * NBUF
    for b in range(NBUF):
        wait_gather(b)
        fire_write(last + b, b)
    for b in range(NBUF):
        wait_write(b)


def kernel(encoded_input, table):
    idx = encoded_input.reshape(-1).astype(jnp.int32).reshape(NW, NCHUNK, CHUNK)
    table_p = jnp.pad(table, ((0, 0), (0, PADDED_DIM - EMBED_DIM)))
    out = _gather_kernel(table_p, idx)
    return out.reshape(BATCH, HIST, EMBED_DIM)


# speedup vs baseline: 1.9433x; 1.0481x over previous
import functools

import jax
import jax.numpy as jnp
from jax import lax
from jax.experimental import pallas as pl
from jax.experimental.pallas import tpu as pltpu
from jax.experimental.pallas import tpu_sc as plsc

VOCAB = 1_000_000
EMBED_DIM = 64
PADDED_DIM = 128
BATCH = 4096
HIST = 50
N = BATCH * HIST

_info = plsc.get_sparse_core_info()
NC = _info.num_cores
NS = _info.num_subcores
NW = NC * NS
PER_W = N // NW
CHUNK = 128
NCHUNK = PER_W // CHUNK
NBUF = 5
NITER = NCHUNK // NBUF

_mesh = plsc.VectorSubcoreMesh(core_axis_name="c", subcore_axis_name="s")


@functools.partial(
    pl.kernel,
    mesh=_mesh,
    compiler_params=pltpu.CompilerParams(use_tc_tiling_on_sc=False),
    out_type=jax.ShapeDtypeStruct((N, EMBED_DIM), jnp.float32),
    scratch_types=(
        [
            pltpu.VMEM((NCHUNK, CHUNK), jnp.int32),
            pltpu.VMEM((NBUF, CHUNK, EMBED_DIM), jnp.float32),
        ]
        + [pltpu.SemaphoreType.DMA] * (2 * NBUF)
    ),
)
def _gather_kernel(table_hbm, idx_hbm, out_hbm, idx_v, rows_v, *sems):
    gsem = sems[:NBUF]
    wsem = sems[NBUF:]
    wid = lax.axis_index("s") * NC + lax.axis_index("c")
    base = wid * PER_W
    pltpu.sync_copy(idx_hbm.at[wid], idx_v)

    def fire_gather(c, b):
        pltpu.async_copy(table_hbm.at[idx_v.at[c]], rows_v.at[b], gsem[b])

    def wait_gather(b):
        pltpu.make_async_copy(table_hbm.at[idx_v.at[0]], rows_v.at[b], gsem[b]).wait()

    def fire_write(c, b):
        pltpu.async_copy(
            rows_v.at[b], out_hbm.at[pl.ds(base + c * CHUNK, CHUNK)], wsem[b]
        )

    def wait_write(b):
        pltpu.make_async_copy(
            rows_v.at[b], out_hbm.at[pl.ds(base, CHUNK)], wsem[b]
        ).wait()

    for b in range(NBUF):
        fire_gather(b, b)

    def cycle(i, carry):
        for b in range(NBUF):
            wait_gather(b)
            fire_write(i * NBUF + b, b)
        for b in range(NBUF):
            wait_write(b)
            fire_gather((i + 1) * NBUF + b, b)
        return carry

    lax.fori_loop(0, NITER - 1, cycle, 0)

    last = (NITER - 1) * NBUF
    for b in range(NBUF):
        wait_gather(b)
        fire_write(last + b, b)
    for b in range(NBUF):
        wait_write(b)


def kernel(encoded_input, table):
    idx = encoded_input.reshape(-1).astype(jnp.int32)
    idx2 = (2 * idx).reshape(NW, NCHUNK, CHUNK)
    table_p = jnp.pad(table, ((0, 0), (0, PADDED_DIM - EMBED_DIM)))
    table_v = table_p.reshape(2 * VOCAB, EMBED_DIM)
    out = _gather_kernel(table_v, idx2)
    return out.reshape(BATCH, HIST, EMBED_DIM)
